# baseline (speedup 1.0000x reference)
import jax
import jax.numpy as jnp
from jax.experimental import pallas as pl
from jax.experimental.pallas import tpu as pltpu

B, L, T, KH, DH = 16, 2, 2048, 8, 64
ROW = 2 * KH * DH
P = 64
Q = (B * L * T) // P


def _body(layer_ref, idx_ref, kv_ref, buf_any, out_any, sem):
    del buf_any
    layer = layer_ref[0]
    idx = idx_ref[0]
    for b in range(B):
        pair = (b * L + layer) * T + idx
        pln = pair // Q
        r = pair - pln * Q
        pltpu.make_async_copy(kv_ref.at[b], out_any.at[pln, r], sem).start()
    for b in range(B):
        pair = (b * L + layer) * T + idx
        pln = pair // Q
        r = pair - pln * Q
        pltpu.make_async_copy(kv_ref.at[b], out_any.at[pln, r], sem).wait()


@jax.jit
def _run(layer_s, idx_s, kv, buf3):
    return pl.pallas_call(
        _body,
        in_specs=[
            pl.BlockSpec(memory_space=pltpu.SMEM),
            pl.BlockSpec(memory_space=pltpu.SMEM),
            pl.BlockSpec(memory_space=pltpu.VMEM),
            pl.BlockSpec(memory_space=pl.ANY),
        ],
        out_specs=pl.BlockSpec(memory_space=pl.ANY),
        out_shape=jax.ShapeDtypeStruct((P, Q, ROW), jnp.float32),
        scratch_shapes=[pltpu.SemaphoreType.DMA],
        input_output_aliases={3: 0},
    )(layer_s, idx_s, kv, buf3)


def kernel(buf, k_step, v_step, layer, idx):
    layer = jnp.clip(jnp.asarray(layer, jnp.int32), 0, L - 1)
    idx = jnp.clip(jnp.asarray(idx, jnp.int32), 0, T - 1)
    kv = jnp.concatenate(
        [k_step.reshape(B, KH * DH), v_step.reshape(B, KH * DH)], axis=1
    )
    out3 = _run(layer.reshape(1), idx.reshape(1), kv, buf.reshape(P, Q, ROW))
    return out3.reshape(B, L, T, 2, KH, DH)


# speedup vs baseline: 1.7846x; 1.7846x over previous
import jax
import jax.numpy as jnp
from jax.experimental import pallas as pl
from jax.experimental.pallas import tpu as pltpu

B, L, T, KH, DH = 16, 2, 2048, 8, 64


def _body(layer_ref, idx_ref, k_ref, v_ref, buf_any, out_any, sem):
    del buf_any
    layer = layer_ref[0]
    idx = idx_ref[0]
    for b in range(B):
        pltpu.make_async_copy(
            k_ref.at[b, 0], out_any.at[b, layer, idx, 0], sem
        ).start()
        pltpu.make_async_copy(
            v_ref.at[b, 0], out_any.at[b, layer, idx, 1], sem
        ).start()
    for b in range(B):
        pltpu.make_async_copy(
            k_ref.at[b, 0], out_any.at[b, layer, idx, 0], sem
        ).wait()
        pltpu.make_async_copy(
            v_ref.at[b, 0], out_any.at[b, layer, idx, 1], sem
        ).wait()


@jax.jit
def _run(layer_s, idx_s, k_step, v_step, buf):
    return pl.pallas_call(
        _body,
        in_specs=[
            pl.BlockSpec(memory_space=pltpu.SMEM),
            pl.BlockSpec(memory_space=pltpu.SMEM),
            pl.BlockSpec(memory_space=pltpu.VMEM),
            pl.BlockSpec(memory_space=pltpu.VMEM),
            pl.BlockSpec(memory_space=pl.ANY),
        ],
        out_specs=pl.BlockSpec(memory_space=pl.ANY),
        out_shape=jax.ShapeDtypeStruct((B, L, T, 2, KH, DH), jnp.float32),
        scratch_shapes=[pltpu.SemaphoreType.DMA],
        input_output_aliases={4: 0},
    )(layer_s, idx_s, k_step, v_step, buf)


def kernel(buf, k_step, v_step, layer, idx):
    layer = jnp.clip(jnp.asarray(layer, jnp.int32), 0, L - 1)
    idx = jnp.clip(jnp.asarray(idx, jnp.int32), 0, T - 1)
    return _run(layer.reshape(1), idx.reshape(1), k_step, v_step, buf)


# speedup vs baseline: 2.9061x; 1.6284x over previous
import jax
import jax.numpy as jnp
from jax.experimental import pallas as pl
from jax.experimental.pallas import tpu as pltpu

B, L, T, KH, DH = 16, 2, 2048, 8, 64
ROW = 2 * KH * DH


def _body(layer_ref, idx_ref, kv_ref, buf_any, out_any, sem):
    del buf_any
    layer = layer_ref[0]
    idx = idx_ref[0]
    for b in range(B):
        pltpu.make_async_copy(
            kv_ref.at[b], out_any.at[b * L + layer, idx], sem
        ).start()
    for b in range(B):
        pltpu.make_async_copy(
            kv_ref.at[b], out_any.at[b * L + layer, idx], sem
        ).wait()


@jax.jit
def _run(layer_s, idx_s, kv, buf3):
    return pl.pallas_call(
        _body,
        in_specs=[
            pl.BlockSpec(memory_space=pltpu.SMEM),
            pl.BlockSpec(memory_space=pltpu.SMEM),
            pl.BlockSpec(memory_space=pltpu.VMEM),
            pl.BlockSpec(memory_space=pl.ANY),
        ],
        out_specs=pl.BlockSpec(memory_space=pl.ANY),
        out_shape=jax.ShapeDtypeStruct((B * L, T, ROW), jnp.float32),
        scratch_shapes=[pltpu.SemaphoreType.DMA],
        input_output_aliases={3: 0},
    )(layer_s, idx_s, kv, buf3)


def kernel(buf, k_step, v_step, layer, idx):
    layer = jnp.clip(jnp.asarray(layer, jnp.int32), 0, L - 1)
    idx = jnp.clip(jnp.asarray(idx, jnp.int32), 0, T - 1)
    kv = jnp.concatenate(
        [k_step.reshape(B, KH * DH), v_step.reshape(B, KH * DH)], axis=1
    )
    out3 = _run(layer.reshape(1), idx.reshape(1), kv, buf.reshape(B * L, T, ROW))
    return out3.reshape(B, L, T, 2, KH, DH)
